# Initial kernel scaffold; baseline (speedup 1.0000x reference)
import jax
import jax.numpy as jnp
from jax.experimental import pallas as pl


def kernel(x, edge_index, edge_attr, batch, y, params):
    raise NotImplementedError("write your pallas kernel here")


# speedup vs baseline: 6.6972x; 6.6972x over previous
import functools

import jax
import jax.numpy as jnp
from jax.experimental import pallas as pl
from jax.experimental.pallas import tpu as pltpu
from jax.experimental.pallas import tpu_sc as plsc

_EPS = 1e-5


def _softplus(x):
    return jnp.maximum(x, 0.0) + jnp.log1p(jnp.exp(-jnp.abs(x)))


def _sigmoid(x):
    return 1.0 / (1.0 + jnp.exp(-x))


def _tile(total, target, mult=8):
    best = None
    for d in range(1, int(total**0.5) + 1):
        if total % d == 0:
            for c in (d, total // d):
                if c <= target and c % mult == 0 and (best is None or c > best):
                    best = c
    if best is None:
        raise ValueError(f"no tile for {total} <= {target} mult {mult}")
    return best


def _embed(x, emb_pad, offs):
    n, f = x.shape
    k, atom = emb_pad.shape
    r = _tile(n, 1024)

    def body(x_ref, t_ref, off_ref, o_ref):
        idx = x_ref[...] + off_ref[...]
        iota = jax.lax.broadcasted_iota(jnp.int32, (r, k), 1)
        oh = jnp.zeros((r, k), jnp.float32)
        for j in range(f):
            oh = oh + (iota == idx[:, j : j + 1]).astype(jnp.float32)
        o_ref[...] = jnp.dot(oh, t_ref[...], preferred_element_type=jnp.float32)

    return pl.pallas_call(
        body,
        grid=(n // r,),
        in_specs=[
            pl.BlockSpec((r, f), lambda i: (i, 0)),
            pl.BlockSpec((k, atom), lambda i: (0, 0)),
            pl.BlockSpec((1, f), lambda i: (0, 0)),
        ],
        out_specs=pl.BlockSpec((r, atom), lambda i: (i, 0)),
        out_shape=jax.ShapeDtypeStruct((n, atom), jnp.float32),
    )(x, emb_pad, offs)


def _prep_uv(h, w2):
    n, atom = h.shape
    cols = w2.shape[2]
    r = _tile(n, 2048)
    g = n // r

    def body(h_ref, w_ref, o_ref):
        o_ref[...] = jnp.dot(
            h_ref[...], w_ref[0], preferred_element_type=jnp.float32
        )

    return pl.pallas_call(
        body,
        grid=(2, g),
        in_specs=[
            pl.BlockSpec((r, atom), lambda j, i: (i, 0)),
            pl.BlockSpec((1, atom, cols), lambda j, i: (j, 0, 0)),
        ],
        out_specs=pl.BlockSpec((r, cols), lambda j, i: (j * g + i, 0)),
        out_shape=jax.ShapeDtypeStruct((2 * n, cols), jnp.float32),
    )(h, w2)


def _gather_rows(table, idx2):
    b = idx2.shape[1]
    d = table.shape[1]
    w = _tile(b, 128)
    mesh = plsc.VectorSubcoreMesh(core_axis_name="c", subcore_axis_name="s")

    @functools.partial(
        pl.kernel,
        out_type=jax.ShapeDtypeStruct((b, d), table.dtype),
        mesh=mesh,
    )
    def gk(t_hbm, i_hbm, o_hbm):
        def body(i_vmem, o_vmem):
            pltpu.sync_copy(t_hbm.at[i_vmem.at[0]], o_vmem)

        pltpu.emit_pipeline(
            body,
            grid=(b // w,),
            in_specs=[pl.BlockSpec((1, w), lambda i: (0, i))],
            out_specs=[pl.BlockSpec((w, d), lambda i: (i, 0))],
            core_axis_name=("c", "s"),
            dimension_semantics=(pltpu.PARALLEL,),
        )(i_hbm, o_hbm)

    return gk(table, idx2)


def _conv_edge(cat, ea, w_ea, b_row):
    e, nbrf = ea.shape
    cols = w_ea.shape[1]
    ta = _tile(e, 6144)
    g_blocks = e // ta

    def body(nbr_ref, ini_ref, ea_ref, w_ref, b_ref, g_ref, s_ref, q_ref):
        g = jnp.dot(ea_ref[...], w_ref[...], preferred_element_type=jnp.float32)
        g = g + nbr_ref[...] + ini_ref[...] + b_ref[...]
        g_ref[...] = g
        rs = jnp.sum(g, axis=0, keepdims=True)
        rq = jnp.sum(g * g, axis=0, keepdims=True)
        i = pl.program_id(0)

        @pl.when(i == 0)
        def _():
            s_ref[...] = rs
            q_ref[...] = rq

        @pl.when(i > 0)
        def _():
            s_ref[...] += rs
            q_ref[...] += rq

    return pl.pallas_call(
        body,
        grid=(g_blocks,),
        in_specs=[
            pl.BlockSpec((ta, cols), lambda i: (i, 0)),
            pl.BlockSpec((ta, cols), lambda i: (i + g_blocks, 0)),
            pl.BlockSpec((ta, nbrf), lambda i: (i, 0)),
            pl.BlockSpec((nbrf, cols), lambda i: (0, 0)),
            pl.BlockSpec((1, cols), lambda i: (0, 0)),
        ],
        out_specs=[
            pl.BlockSpec((ta, cols), lambda i: (i, 0)),
            pl.BlockSpec((1, cols), lambda i: (0, 0)),
            pl.BlockSpec((1, cols), lambda i: (0, 0)),
        ],
        out_shape=[
            jax.ShapeDtypeStruct((e, cols), jnp.float32),
            jax.ShapeDtypeStruct((1, cols), jnp.float32),
            jax.ShapeDtypeStruct((1, cols), jnp.float32),
        ],
    )(cat, cat, ea, w_ea, b_row)


def _conv_actred(g3, sc1, sh1):
    n, deg, cols = g3.shape
    atom = cols // 2
    rb = _tile(n, 512)

    def body(g_ref, sc_ref, sh_ref, s_ref, a_ref, q_ref):
        g = g_ref[...] * sc_ref[...] + sh_ref[...]
        filt = g[:, :, 0:atom]
        core = g[:, :, atom:cols]
        act = _sigmoid(filt) * _softplus(core)
        s = jnp.sum(act, axis=1)
        s_ref[...] = s
        ra = jnp.sum(s, axis=0, keepdims=True)
        rq = jnp.sum(s * s, axis=0, keepdims=True)
        i = pl.program_id(0)

        @pl.when(i == 0)
        def _():
            a_ref[...] = ra
            q_ref[...] = rq

        @pl.when(i > 0)
        def _():
            a_ref[...] += ra
            q_ref[...] += rq

    return pl.pallas_call(
        body,
        grid=(n // rb,),
        in_specs=[
            pl.BlockSpec((rb, deg, cols), lambda i: (i, 0, 0)),
            pl.BlockSpec((1, cols), lambda i: (0, 0)),
            pl.BlockSpec((1, cols), lambda i: (0, 0)),
        ],
        out_specs=[
            pl.BlockSpec((rb, atom), lambda i: (i, 0)),
            pl.BlockSpec((1, atom), lambda i: (0, 0)),
            pl.BlockSpec((1, atom), lambda i: (0, 0)),
        ],
        out_shape=[
            jax.ShapeDtypeStruct((n, atom), jnp.float32),
            jax.ShapeDtypeStruct((1, atom), jnp.float32),
            jax.ShapeDtypeStruct((1, atom), jnp.float32),
        ],
    )(g3, sc1, sh1)


def _resid(h, s, sc2, sh2):
    n, atom = h.shape
    rc = _tile(n, 2048)

    def body(h_ref, s_ref, sc_ref, sh_ref, o_ref):
        o_ref[...] = _softplus(h_ref[...] + s_ref[...] * sc_ref[...] + sh_ref[...])

    return pl.pallas_call(
        body,
        grid=(n // rc,),
        in_specs=[
            pl.BlockSpec((rc, atom), lambda i: (i, 0)),
            pl.BlockSpec((rc, atom), lambda i: (i, 0)),
            pl.BlockSpec((1, atom), lambda i: (0, 0)),
            pl.BlockSpec((1, atom), lambda i: (0, 0)),
        ],
        out_specs=pl.BlockSpec((rc, atom), lambda i: (i, 0)),
        out_shape=jax.ShapeDtypeStruct((n, atom), jnp.float32),
    )(h, s, sc2, sh2)


def _pool(h, b3, ngraph):
    n, atom = h.shape
    rp = b3.shape[2]

    def body(h_ref, b_ref, p_ref, c_ref):
        bv = b_ref[0]
        iota = jax.lax.broadcasted_iota(jnp.int32, (ngraph, rp), 0)
        oh = (iota == bv).astype(jnp.float32)
        ps = jnp.dot(oh, h_ref[...], preferred_element_type=jnp.float32)
        cs = jnp.sum(oh, axis=1, keepdims=True)
        i = pl.program_id(0)

        @pl.when(i == 0)
        def _():
            p_ref[...] = ps
            c_ref[...] = cs

        @pl.when(i > 0)
        def _():
            p_ref[...] += ps
            c_ref[...] += cs

    return pl.pallas_call(
        body,
        grid=(n // rp,),
        in_specs=[
            pl.BlockSpec((rp, atom), lambda i: (i, 0)),
            pl.BlockSpec((1, 1, rp), lambda i: (i, 0, 0)),
        ],
        out_specs=[
            pl.BlockSpec((ngraph, atom), lambda i: (0, 0)),
            pl.BlockSpec((ngraph, 1), lambda i: (0, 0)),
        ],
        out_shape=[
            jax.ShapeDtypeStruct((ngraph, atom), jnp.float32),
            jax.ShapeDtypeStruct((ngraph, 1), jnp.float32),
        ],
    )(h, b3)


def _head(pool, cnt, bn_g, bn_b, w1, b1, wo_row, bo):
    ngraph, atom = pool.shape

    def body(p_ref, c_ref, g_ref, b_ref, w1_ref, b1_ref, wo_ref, bo_ref, o_ref):
        pooled = p_ref[...] / jnp.maximum(c_ref[...], 1.0)
        z = _softplus(pooled)
        m = jnp.mean(z, axis=0, keepdims=True)
        v = jnp.mean((z - m) ** 2, axis=0, keepdims=True)
        z = (z - m) / jnp.sqrt(v + _EPS) * g_ref[...] + b_ref[...]
        z2 = jnp.dot(z, w1_ref[...], preferred_element_type=jnp.float32) + b1_ref[...]
        z2 = _softplus(z2)
        o_ref[...] = jnp.sum(z2 * wo_ref[...], axis=1, keepdims=True) + bo_ref[...]

    return pl.pallas_call(
        body,
        out_shape=jax.ShapeDtypeStruct((ngraph, 1), jnp.float32),
    )(pool, cnt, bn_g, bn_b, w1, b1, wo_row, bo)


def kernel(x, edge_index, edge_attr, batch, y, params):
    n = x.shape[0]
    e, nbrf = edge_attr.shape
    deg = e // n
    emb = params["emb"]
    atom = emb[0].shape[1]
    ngraph = y.shape[0]

    dims = [t.shape[0] for t in emb]
    total = sum(dims)
    kpad = -(-total // 64) * 64
    table = jnp.concatenate(emb, axis=0)
    table = jnp.pad(table, ((0, kpad - total), (0, 0)))
    offs = []
    acc = 0
    for dsz in dims:
        offs.append(acc)
        acc += dsz
    offs = jnp.asarray(offs, jnp.int32).reshape(1, len(dims))

    h = _embed(x, table, offs)

    idx2 = jnp.concatenate([edge_index[1], edge_index[0] + n]).reshape(1, 2 * e)

    for c in params["convs"]:
        w = c["W"]
        w2 = jnp.stack([w[0:atom], w[atom : 2 * atom]])
        w_ea = w[2 * atom :]
        b_row = c["b"].reshape(1, 2 * atom)

        uv = _prep_uv(h, w2)
        cat = _gather_rows(uv, idx2)
        g, gs, gq = _conv_edge(cat, edge_attr, w_ea, b_row)
        m1 = gs / e
        v1 = gq / e - m1 * m1
        sc1 = c["g1"].reshape(1, 2 * atom) * jax.lax.rsqrt(v1 + _EPS)
        sh1 = c["b1"].reshape(1, 2 * atom) - m1 * sc1

        g3 = g.reshape(n, deg, 2 * atom)
        s, sa, sq = _conv_actred(g3, sc1, sh1)
        m2 = sa / n
        v2 = sq / n - m2 * m2
        sc2 = c["g2"].reshape(1, atom) * jax.lax.rsqrt(v2 + _EPS)
        sh2 = c["b2"].reshape(1, atom) - m2 * sc2

        h = _resid(h, s, sc2, sh2)

    rp = _tile(n, 1024)
    b3 = batch.reshape(n // rp, 1, rp)
    pool, cnt = _pool(h, b3, ngraph)

    out = _head(
        pool,
        cnt,
        params["bn_g"].reshape(1, atom),
        params["bn_b"].reshape(1, atom),
        params["W1"],
        params["b1"].reshape(1, params["W1"].shape[1]),
        params["Wo"].reshape(1, params["W1"].shape[1]),
        params["bo"].reshape(1, 1),
    )
    return out
